# initial kernel scaffold (unmeasured)
import jax
import jax.numpy as jnp
from jax import lax
from jax.experimental import pallas as pl
from jax.experimental.pallas import tpu as pltpu

N_DEV = 4
SCALE = 128 ** -0.5


def _local_flash_body(q_ref, k_ref, v_ref, o_ref, m_ref, l_ref):
    q = q_ref[0, :, 0, :]
    k = k_ref[0, :, 0, :]
    v = v_ref[0, :, 0, :]
    s = lax.dot_general(
        q, k, (((1,), (1,)), ((), ())), preferred_element_type=jnp.float32
    ) * SCALE
    m = jnp.max(s, axis=-1, keepdims=True)
    p = jnp.exp(s - m)
    l = jnp.sum(p, axis=-1, keepdims=True)
    o = lax.dot_general(
        p, v, (((1,), (0,)), ((), ())), preferred_element_type=jnp.float32
    )
    o_ref[0, :, 0, :] = o
    m_ref[0, :, 0] = m[:, 0]
    l_ref[0, :, 0] = l[:, 0]


def _allreduce_body(
    o_ref, m_ref, l_ref, out_ref, o_comm, s_comm, o_ssem, o_rsem, s_ssem, s_rsem
):
    my = lax.axis_index("i")
    left = (my - 1) % N_DEV
    right = (my + 1) % N_DEV

    barrier = pltpu.get_barrier_semaphore()
    for nbr in (left, right):
        pl.semaphore_signal(
            barrier, inc=1, device_id=(nbr,), device_id_type=pl.DeviceIdType.MESH
        )
    pl.semaphore_wait(barrier, 2)

    o_comm[0] = o_ref[...]
    s_comm[0, 0] = m_ref[...]
    s_comm[0, 1] = l_ref[...]

    for h in range(N_DEV - 1):
        ro = pltpu.make_async_remote_copy(
            src_ref=o_comm.at[h],
            dst_ref=o_comm.at[h + 1],
            send_sem=o_ssem.at[h],
            recv_sem=o_rsem.at[h],
            device_id=(right,),
            device_id_type=pl.DeviceIdType.MESH,
        )
        rs = pltpu.make_async_remote_copy(
            src_ref=s_comm.at[h],
            dst_ref=s_comm.at[h + 1],
            send_sem=s_ssem.at[h],
            recv_sem=s_rsem.at[h],
            device_id=(right,),
            device_id_type=pl.DeviceIdType.MESH,
        )
        ro.start()
        rs.start()
        ro.wait()
        rs.wait()

    m_g = s_comm[0, 0]
    for k in range(1, N_DEV):
        m_g = jnp.maximum(m_g, s_comm[k, 0])
    w = jnp.exp(s_comm[0, 0] - m_g)
    l_g = s_comm[0, 1] * w
    o_g = o_comm[0] * w[..., None]
    for k in range(1, N_DEV):
        w = jnp.exp(s_comm[k, 0] - m_g)
        l_g = l_g + s_comm[k, 1] * w
        o_g = o_g + o_comm[k] * w[..., None]
    out_ref[...] = o_g / l_g[..., None]


def kernel(Q, K, V):
    b, sq, h, d = Q.shape
    kv = K.shape[1]

    o_part, m_part, l_part = pl.pallas_call(
        _local_flash_body,
        grid=(b, h),
        in_specs=[
            pl.BlockSpec((1, sq, 1, d), lambda i, j: (i, 0, j, 0)),
            pl.BlockSpec((1, kv, 1, d), lambda i, j: (i, 0, j, 0)),
            pl.BlockSpec((1, kv, 1, d), lambda i, j: (i, 0, j, 0)),
        ],
        out_specs=[
            pl.BlockSpec((1, sq, 1, d), lambda i, j: (i, 0, j, 0)),
            pl.BlockSpec((1, sq, 1), lambda i, j: (i, 0, j)),
            pl.BlockSpec((1, sq, 1), lambda i, j: (i, 0, j)),
        ],
        out_shape=[
            jax.ShapeDtypeStruct((b, sq, h, d), jnp.float32),
            jax.ShapeDtypeStruct((b, sq, h), jnp.float32),
            jax.ShapeDtypeStruct((b, sq, h), jnp.float32),
        ],
    )(Q, K, V)

    return pl.pallas_call(
        _allreduce_body,
        out_shape=jax.ShapeDtypeStruct((b, sq, h, d), jnp.float32),
        in_specs=[pl.BlockSpec(memory_space=pltpu.VMEM)] * 3,
        out_specs=pl.BlockSpec(memory_space=pltpu.VMEM),
        scratch_shapes=[
            pltpu.VMEM((N_DEV, b, sq, h, d), jnp.float32),
            pltpu.VMEM((N_DEV, 2, b, sq, h), jnp.float32),
            pltpu.SemaphoreType.DMA((N_DEV - 1,)),
            pltpu.SemaphoreType.DMA((N_DEV - 1,)),
            pltpu.SemaphoreType.DMA((N_DEV - 1,)),
            pltpu.SemaphoreType.DMA((N_DEV - 1,)),
        ],
        compiler_params=pltpu.CompilerParams(collective_id=0),
    )(o_part, m_part, l_part)


# baseline (device time: 155067 ns/iter reference)
import jax
import jax.numpy as jnp
from jax import lax
from jax.experimental import pallas as pl
from jax.experimental.pallas import tpu as pltpu

N_DEV = 4
SCALE = 128 ** -0.5


NEG_INF = -1e30


def _local_flash_body(q_ref, k_ref, v_ref, o_ref, m_ref, l_ref, macc, lacc, oacc):
    ki = pl.program_id(1)
    nk = pl.num_programs(1)
    n_heads = q_ref.shape[2]

    @pl.when(ki == 0)
    def _():
        macc[...] = jnp.full_like(macc, NEG_INF)
        lacc[...] = jnp.zeros_like(lacc)
        oacc[...] = jnp.zeros_like(oacc)

    for j in range(n_heads):
        q = q_ref[0, :, j, :]
        k = k_ref[0, :, j, :]
        v = v_ref[0, :, j, :]
        s = lax.dot_general(
            q, k, (((1,), (1,)), ((), ())), preferred_element_type=jnp.float32
        ) * SCALE
        m_old = macc[:, j : j + 1]
        m_cur = jnp.maximum(m_old, jnp.max(s, axis=-1, keepdims=True))
        p = jnp.exp(s - m_cur)
        alpha = jnp.exp(m_old - m_cur)
        pv = lax.dot_general(
            p, v, (((1,), (0,)), ((), ())), preferred_element_type=jnp.float32
        )
        lacc[:, j : j + 1] = lacc[:, j : j + 1] * alpha + jnp.sum(
            p, axis=-1, keepdims=True
        )
        oacc[:, j, :] = oacc[:, j, :] * alpha + pv
        macc[:, j : j + 1] = m_cur

    @pl.when(ki == nk - 1)
    def _():
        o_ref[0] = oacc[...]
        m_ref[0] = macc[...]
        l_ref[0] = lacc[...]


def _allreduce_body(
    o_ref, m_ref, l_ref, out_ref, o_comm, s_comm, o_ssem, o_rsem, s_ssem, s_rsem
):
    my = lax.axis_index("i")
    left = (my - 1) % N_DEV
    right = (my + 1) % N_DEV

    barrier = pltpu.get_barrier_semaphore()
    for nbr in (left, right):
        pl.semaphore_signal(
            barrier, inc=1, device_id=(nbr,), device_id_type=pl.DeviceIdType.MESH
        )
    pl.semaphore_wait(barrier, 2)

    o_comm[0] = o_ref[...]
    s_comm[0, 0] = m_ref[...]
    s_comm[0, 1] = l_ref[...]

    for h in range(N_DEV - 1):
        ro = pltpu.make_async_remote_copy(
            src_ref=o_comm.at[h],
            dst_ref=o_comm.at[h + 1],
            send_sem=o_ssem.at[h],
            recv_sem=o_rsem.at[h],
            device_id=(right,),
            device_id_type=pl.DeviceIdType.MESH,
        )
        rs = pltpu.make_async_remote_copy(
            src_ref=s_comm.at[h],
            dst_ref=s_comm.at[h + 1],
            send_sem=s_ssem.at[h],
            recv_sem=s_rsem.at[h],
            device_id=(right,),
            device_id_type=pl.DeviceIdType.MESH,
        )
        ro.start()
        rs.start()
        ro.wait()
        rs.wait()

    m_g = s_comm[0, 0]
    for k in range(1, N_DEV):
        m_g = jnp.maximum(m_g, s_comm[k, 0])
    w = jnp.exp(s_comm[0, 0] - m_g)
    l_g = s_comm[0, 1] * w
    o_g = o_comm[0] * w[..., None]
    for k in range(1, N_DEV):
        w = jnp.exp(s_comm[k, 0] - m_g)
        l_g = l_g + s_comm[k, 1] * w
        o_g = o_g + o_comm[k] * w[..., None]
    out_ref[...] = o_g / l_g[..., None]


def kernel(Q, K, V):
    b, sq, h, d = Q.shape
    kv = K.shape[1]

    kc = 512
    nk = kv // kc

    o_part, m_part, l_part = pl.pallas_call(
        _local_flash_body,
        grid=(b, nk),
        in_specs=[
            pl.BlockSpec((1, sq, h, d), lambda i, ki: (i, 0, 0, 0)),
            pl.BlockSpec((1, kc, h, d), lambda i, ki: (i, ki, 0, 0)),
            pl.BlockSpec((1, kc, h, d), lambda i, ki: (i, ki, 0, 0)),
        ],
        out_specs=[
            pl.BlockSpec((1, sq, h, d), lambda i, ki: (i, 0, 0, 0)),
            pl.BlockSpec((1, sq, h), lambda i, ki: (i, 0, 0)),
            pl.BlockSpec((1, sq, h), lambda i, ki: (i, 0, 0)),
        ],
        out_shape=[
            jax.ShapeDtypeStruct((b, sq, h, d), jnp.float32),
            jax.ShapeDtypeStruct((b, sq, h), jnp.float32),
            jax.ShapeDtypeStruct((b, sq, h), jnp.float32),
        ],
        scratch_shapes=[
            pltpu.VMEM((sq, h), jnp.float32),
            pltpu.VMEM((sq, h), jnp.float32),
            pltpu.VMEM((sq, h, d), jnp.float32),
        ],
    )(Q, K, V)

    return pl.pallas_call(
        _allreduce_body,
        out_shape=jax.ShapeDtypeStruct((b, sq, h, d), jnp.float32),
        in_specs=[pl.BlockSpec(memory_space=pltpu.VMEM)] * 3,
        out_specs=pl.BlockSpec(memory_space=pltpu.VMEM),
        scratch_shapes=[
            pltpu.VMEM((N_DEV, b, sq, h, d), jnp.float32),
            pltpu.VMEM((N_DEV, 2, b, sq, h), jnp.float32),
            pltpu.SemaphoreType.DMA((N_DEV - 1,)),
            pltpu.SemaphoreType.DMA((N_DEV - 1,)),
            pltpu.SemaphoreType.DMA((N_DEV - 1,)),
            pltpu.SemaphoreType.DMA((N_DEV - 1,)),
        ],
        compiler_params=pltpu.CompilerParams(collective_id=0),
    )(o_part, m_part, l_part)
